# Initial kernel scaffold; baseline (speedup 1.0000x reference)
import jax
import jax.numpy as jnp
from jax.experimental import pallas as pl


def kernel(user_embs, pos_embs, neg_embs, hard_negatives, loss_mask, item_table):
    raise NotImplementedError("write your pallas kernel here")


# speedup vs baseline: 11.1187x; 11.1187x over previous
import functools

import jax
import jax.numpy as jnp
from jax import lax
from jax.experimental import pallas as pl
from jax.experimental.pallas import tpu as pltpu
from jax.experimental.pallas import tpu_sc as plsc

D = 64
KH = 32
NUM_INB = 359
INB_PAD = 384
NLANE = 16
CHUNK = 16
NW = 32
INV_TEMP = 20.0
SHIFT = 20.0
N_INB_GRP = 23
CONTRASTIVE_W, RANKING_W, DIVERSITY_W = 1.0, 0.1, 0.01


def _norm_body(x_ref, o_ref):
    x = x_ref[...]
    n = jnp.sqrt(jnp.sum(x * x, axis=-1, keepdims=True))
    o_ref[...] = x / jnp.maximum(n, 1e-12)


def _normalize_rows(x, block_rows):
    rows = x.shape[0]
    grid = rows // block_rows
    return pl.pallas_call(
        _norm_body,
        grid=(grid,),
        in_specs=[pl.BlockSpec((block_rows, D), lambda i: (i, 0))],
        out_specs=pl.BlockSpec((block_rows, D), lambda i: (i, 0)),
        out_shape=jax.ShapeDtypeStruct((rows, D), jnp.float32),
    )(x)


def _sc_body(qn, kn, tn, hard, inb, mvec, l0_out, den_out,
             hard_v, inb_v, qc, kc, hrows, irows, l0r, denr, mv, sem):
    wid = lax.axis_index("s") * 2 + lax.axis_index("c")
    pltpu.sync_copy(mvec, mv)
    m = mv[...][0]
    lane = lax.broadcasted_iota(jnp.int32, (NLANE,), 0)
    cnt1 = jnp.where(lane < 9, 5.0, 4.0)
    tail_mask = lane < (NUM_INB - (N_INB_GRP - 1) * NLANE)
    xor_idx = [lane ^ s for s in (1, 2, 4, 8)]
    sel_m = [(lane & s) == 0 for s in (1, 2, 4, 8)]

    def hsum(v):
        for ix in xor_idx:
            v = v + v.at[ix].get(mode="promise_in_bounds")
        return v

    def tree16(regs):
        for st in range(4):
            msk, ix = sel_m[st], xor_idx[st]
            nxt = []
            for i in range(0, len(regs), 2):
                a, b = regs[i], regs[i + 1]
                d = jnp.where(msk, b, a).at[ix].get(mode="promise_in_bounds")
                nxt.append(jnp.where(msk, a, b) + d)
            regs = nxt
        return regs[0]

    nchunks_total = l0_out.shape[0] // NW
    trip = (m - CHUNK * wid + CHUNK * NW - 1) // (CHUNK * NW)
    nchunks = jnp.maximum(jnp.int32(0),
                          jnp.minimum(jnp.int32(nchunks_total), trip))

    def chunk_body(j, _):
        c = wid + NW * j
        base = CHUNK * c
        pltpu.sync_copy(hard.at[pl.ds(base, CHUNK)], hard_v)
        pltpu.sync_copy(inb.at[pl.ds(base, CHUNK)], inb_v)
        pltpu.sync_copy(qn.at[pl.ds(base, CHUNK)], qc)
        pltpu.sync_copy(kn.at[pl.ds(base, CHUNK)], kc)
        vcount = lax.min(jnp.int32(CHUNK), m - base)

        def pos_body(i, carry):
            l0_reg, den_reg = carry
            cph = pltpu.async_copy(tn.at[hard_v.at[i]], hrows, sem)
            cp0 = pltpu.async_copy(kn.at[inb_v.at[i, 0]],
                                   irows.at[pl.ds(0, 128)], sem)
            cp1 = pltpu.async_copy(kn.at[inb_v.at[i, 1]],
                                   irows.at[pl.ds(128, 128)], sem)
            cp2 = pltpu.async_copy(kn.at[inb_v.at[i, 2]],
                                   irows.at[pl.ds(256, 128)], sem)

            qd = [qc[i, pl.ds(g * NLANE, NLANE)] for g in range(4)]
            kd = [kc[i, pl.ds(g * NLANE, NLANE)] for g in range(4)]
            p = qd[0] * kd[0]
            for g in range(1, 4):
                p = p + qd[g] * kd[g]
            l0v = hsum(p) * INV_TEMP - SHIFT
            l0_reg = jnp.where(lane == i, l0v, l0_reg)

            cph.wait()
            cp0.wait()
            cp1.wait()
            cp2.wait()

            def group(ref, base_row):
                regs = []
                for r in range(NLANE):
                    pr = ref[base_row + r, pl.ds(0, NLANE)] * qd[0]
                    for g in range(1, 4):
                        pr = pr + (ref[base_row + r, pl.ds(g * NLANE, NLANE)]
                                   * qd[g])
                    regs.append(pr)
                return tree16(regs)

            dacc = 5.0 * jnp.exp(group(hrows, 0) * INV_TEMP - SHIFT)
            dacc = dacc + cnt1 * jnp.exp(group(hrows, 16) * INV_TEMP - SHIFT)

            def inb_body(g, acc):
                t = group(irows, g * NLANE)
                return acc + jnp.exp(t * INV_TEMP - SHIFT)

            dacc = lax.fori_loop(0, N_INB_GRP - 1, inb_body, dacc)
            t = group(irows, (N_INB_GRP - 1) * NLANE)
            dacc = dacc + jnp.where(tail_mask,
                                    jnp.exp(t * INV_TEMP - SHIFT), 0.0)

            den = hsum(dacc) + jnp.exp(l0v)
            den_reg = jnp.where(lane == i, den, den_reg)
            return l0_reg, den_reg

        z = jnp.zeros((NLANE,), jnp.float32)
        l0_reg, den_reg = lax.fori_loop(0, vcount, pos_body, (z, z))
        l0r[...] = l0_reg
        denr[...] = den_reg
        pltpu.sync_copy(l0r, l0_out.at[c])
        pltpu.sync_copy(denr, den_out.at[c])
        return 0

    lax.fori_loop(0, nchunks, chunk_body, 0)


def _make_sc_kernel(n):
    nchunks = n // CHUNK
    mesh = plsc.VectorSubcoreMesh(core_axis_name="c", subcore_axis_name="s")
    return functools.partial(
        pl.kernel,
        out_type=[jax.ShapeDtypeStruct((nchunks, CHUNK), jnp.float32),
                  jax.ShapeDtypeStruct((nchunks, CHUNK), jnp.float32)],
        mesh=mesh,
        compiler_params=pltpu.CompilerParams(use_tc_tiling_on_sc=False),
        scratch_types=[
            pltpu.VMEM((CHUNK, KH), jnp.int32),
            pltpu.VMEM((CHUNK, 3, 128), jnp.int32),
            pltpu.VMEM((CHUNK, D), jnp.float32),
            pltpu.VMEM((CHUNK, D), jnp.float32),
            pltpu.VMEM((KH, D), jnp.float32),
            pltpu.VMEM((INB_PAD, D), jnp.float32),
            pltpu.VMEM((CHUNK,), jnp.float32),
            pltpu.VMEM((CHUNK,), jnp.float32),
            pltpu.VMEM((NLANE,), jnp.int32),
            pltpu.SemaphoreType.DMA,
        ],
    )(_sc_body)


def _tail_body(m_ref, l0_ref, den_ref, ue_ref, pe_ref, ne_ref, lm_ref, o_ref):
    m = m_ref[0, 0]
    mf = m.astype(jnp.float32)
    rows, cols = l0_ref.shape
    r = lax.broadcasted_iota(jnp.int32, (rows, cols), 0)
    c = lax.broadcasted_iota(jnp.int32, (rows, cols), 1)
    p = r * cols + c
    per = jnp.where(p < m, jnp.log(den_ref[...]) - l0_ref[...], 0.0)
    contrastive = jnp.sum(per) / mf

    ue = ue_ref[...]
    x = jnp.sum(ue * (pe_ref[...] - ne_ref[...]), axis=-1, keepdims=True)
    lm = lm_ref[...]
    ranking = jnp.sum(-jax.nn.log_sigmoid(x) * lm) / jnp.sum(lm)

    n = jnp.sqrt(jnp.sum(ue * ue, axis=-1, keepdims=True))
    un = ue / jnp.maximum(n, 1e-12)
    sim = lax.dot_general(un, un, (((1,), (1,)), ((), ())),
                          preferred_element_type=jnp.float32)
    b = ue.shape[0]
    ri = lax.broadcasted_iota(jnp.int32, (b, b), 0)
    ci = lax.broadcasted_iota(jnp.int32, (b, b), 1)
    offd = jnp.where(ri == ci, 0.0, jnp.abs(sim))
    diversity = jnp.sum(offd) / (float(b) * float(b))

    o_ref[...] = jnp.full((1, 1), CONTRASTIVE_W * contrastive
                          + RANKING_W * ranking + DIVERSITY_W * diversity)


def _tail(m_count, l0m, denm, ue, pe, ne, lm):
    vspec = pl.BlockSpec(memory_space=pltpu.VMEM)
    return pl.pallas_call(
        _tail_body,
        in_specs=[pl.BlockSpec(memory_space=pltpu.SMEM)] + [vspec] * 6,
        out_specs=vspec,
        out_shape=jax.ShapeDtypeStruct((1, 1), jnp.float32),
    )(m_count, l0m, denm, ue, pe, ne, lm)


def kernel(user_embs, pos_embs, neg_embs, hard_negatives, loss_mask,
           item_table):
    n = loss_mask.shape[0] * loss_mask.shape[1]
    mask_flat = loss_mask.reshape(-1) > 0
    mi = mask_flat.astype(jnp.int32)
    m_count = jnp.sum(mi)
    rank_valid = jnp.cumsum(mi) - mi
    rank_invalid = (jnp.arange(n, dtype=jnp.int32) - rank_valid)
    pos = jnp.where(mask_flat, rank_valid, m_count + rank_invalid)
    order = jnp.zeros((n,), jnp.int32).at[pos].set(
        jnp.arange(n, dtype=jnp.int32))

    q_raw = user_embs.reshape(-1, D)[order]
    k_raw = pos_embs.reshape(-1, D)[order]
    hard_ids = hard_negatives.reshape(-1, KH)[order].astype(jnp.int32)

    r = jax.random.randint(jax.random.key(2), (n, NUM_INB), 0, m_count - 1)
    rows = jnp.arange(n)
    inb = (r + (r >= rows[:, None]).astype(r.dtype)).astype(jnp.int32)
    pad = jnp.broadcast_to(jnp.arange(INB_PAD - NUM_INB, dtype=jnp.int32),
                           (n, INB_PAD - NUM_INB))
    inb = jnp.concatenate([inb, pad], axis=1)

    qn = _normalize_rows(q_raw, 2048)
    kn = _normalize_rows(k_raw, 2048)
    tn = _normalize_rows(item_table, item_table.shape[0] // 10)

    mvec = jnp.full((NLANE,), m_count, jnp.int32)
    l0m, denm = _make_sc_kernel(n)(qn, kn, tn, hard_ids,
                                   inb.reshape(n, 3, 128), mvec)

    out = _tail(jnp.reshape(m_count, (1, 1)),
                l0m.reshape(n // 128, 128), denm.reshape(n // 128, 128),
                user_embs[:, -1, :], pos_embs[:, -1, :], neg_embs[:, -1, :],
                loss_mask[:, -1].reshape(-1, 1))
    return out.reshape(())
